# baseline (speedup 1.0000x reference)
import jax
import jax.numpy as jnp
from jax import lax
from jax.experimental import pallas as pl
from jax.experimental.pallas import tpu as pltpu


def _fw_body(a_ref, o_ref, s_ref):
    n = a_ref.shape[-1]
    a = a_ref[0]
    rows = lax.broadcasted_iota(jnp.int32, (n, n), 0)
    cols = lax.broadcasted_iota(jnp.int32, (n, n), 1)
    eye = rows == cols
    w = jnp.where((a != 0.0) | eye, a, jnp.inf)
    d0 = jnp.where(eye, 0.0, w)

    o_ref[0] = d0

    B = 8

    C = 32

    def do_block(kb, load, store):
        base = kb * B
        p = pltpu.roll(load(pl.ds(base, B)), -base, axis=1)
        for t in range(B):
            p = jnp.minimum(p, p[:, t : t + 1] + p[t : t + 1, :])
        r = pltpu.roll(p, base, axis=1)
        for s in range(n // C):
            d = load(pl.ds(s * C, C))
            c0 = pltpu.roll(d, -base, axis=1)[:, 0:B]
            us = [c0[:, t : t + 1] + r[t : t + 1, :] for t in range(B)]
            while len(us) > 1:
                us = [jnp.minimum(us[i], us[i + 1]) for i in range(0, len(us), 2)]
            store(pl.ds(s * C, C), jnp.minimum(d, us[0]))

    def load_o(ix):
        return o_ref[0, ix, :]

    def store_o(ix, v):
        o_ref[0, ix, :] = v

    def load_s(ix):
        return s_ref[ix, :]

    def store_s(ix, v):
        s_ref[ix, :] = v

    def block_pair(i, _):
        do_block(2 * i, load_o, store_s)
        do_block(2 * i + 1, load_s, store_o)
        return 0

    lax.fori_loop(0, n // (2 * B), block_pair, 0)


def kernel(adj):
    n = adj.shape[-1]
    batch = adj.shape[0] * adj.shape[1]
    a = adj.reshape(batch, n, n)
    out = pl.pallas_call(
        _fw_body,
        grid=(batch,),
        in_specs=[pl.BlockSpec((1, n, n), lambda b: (b, 0, 0))],
        out_specs=pl.BlockSpec((1, n, n), lambda b: (b, 0, 0)),
        out_shape=jax.ShapeDtypeStruct((batch, n, n), adj.dtype),
        scratch_shapes=[pltpu.VMEM((n, n), jnp.float32)],
    )(a)
    return out.reshape(adj.shape)
